# Initial kernel scaffold; baseline (speedup 1.0000x reference)
import jax
import jax.numpy as jnp
from jax.experimental import pallas as pl


def kernel(inputs, hx, time_axis, adp, support, W_ru, W_c):
    raise NotImplementedError("write your pallas kernel here")


# speedup vs baseline: 2.0975x; 2.0975x over previous
import jax
import jax.numpy as jnp
from jax.experimental import pallas as pl
from jax.experimental.pallas import tpu as pltpu

N = 325
NU = 64
IN_DIM = 2
IS = IN_DIM + NU
ALPHA = 0.05
NUM_MAT = 5


def _prep_w(W, out_dim):
    a = ALPHA
    coef = jnp.array([1.0, 1.0, (1 - a) * a, (1 - a) ** 2 * a, (1 - a) ** 3],
                     dtype=jnp.float32)
    Wp = W.reshape(IS, NUM_MAT, out_dim).transpose(1, 0, 2) * coef[:, None, None]
    return Wp.reshape(NUM_MAT * IS, out_dim)


def _body(x1_ref, adp_ref, sup_ref, wru_ref, wc_ref, out_ref):
    adp = adp_ref[0]
    sup = sup_ref[...]
    x1 = x1_ref[0]
    hx = x1[:, IN_DIM:]

    def hops(x):
        s = jnp.dot(sup, x, preferred_element_type=jnp.float32)
        y1 = jnp.dot(adp, x, preferred_element_type=jnp.float32)
        y2 = jnp.dot(adp, y1, preferred_element_type=jnp.float32)
        y3 = jnp.dot(adp, y2, preferred_element_type=jnp.float32)
        return jnp.concatenate([x, s, y1, y2, y3], axis=1)

    ru = jax.nn.sigmoid(
        jnp.dot(hops(x1), wru_ref[...], preferred_element_type=jnp.float32))
    r = ru[:, :NU]
    u = ru[:, NU:]
    x2 = jnp.concatenate([x1[:, :IN_DIM], r * hx], axis=1)
    c = jnp.tanh(
        jnp.dot(hops(x2), wc_ref[...], preferred_element_type=jnp.float32))
    out_ref[0] = u * hx + (1.0 - u) * c


def kernel(inputs, hx, time_axis, adp, support, W_ru, W_c):
    B = inputs.shape[0]
    x1 = jnp.concatenate(
        [inputs.reshape(B, N, IN_DIM), hx.reshape(B, N, NU)], axis=2)
    wru = _prep_w(W_ru, 2 * NU)
    wc = _prep_w(W_c, NU)
    out = pl.pallas_call(
        _body,
        grid=(B,),
        in_specs=[
            pl.BlockSpec((1, N, IS), lambda b: (b, 0, 0)),
            pl.BlockSpec((1, N, N), lambda b: (b, 0, 0)),
            pl.BlockSpec((N, N), lambda b: (0, 0)),
            pl.BlockSpec((NUM_MAT * IS, 2 * NU), lambda b: (0, 0)),
            pl.BlockSpec((NUM_MAT * IS, NU), lambda b: (0, 0)),
        ],
        out_specs=pl.BlockSpec((1, N, NU), lambda b: (b, 0, 0)),
        out_shape=jax.ShapeDtypeStruct((B, N, NU), jnp.float32),
        compiler_params=pltpu.CompilerParams(
            dimension_semantics=("arbitrary",)),
    )(x1, adp, support, wru, wc)
    return out.reshape(B, N * NU)
